# Initial kernel scaffold; baseline (speedup 1.0000x reference)
import jax
import jax.numpy as jnp
from jax.experimental import pallas as pl


def kernel(draft_token_ids, num_draft_tokens, draft_probs, target_probs, bonus_token_ids, uniform_samples):
    raise NotImplementedError("write your pallas kernel here")


# speedup vs baseline: 78.7113x; 78.7113x over previous
import functools

import jax
import jax.numpy as jnp
from jax import lax
from jax.experimental import pallas as pl
from jax.experimental.pallas import tpu as pltpu
from jax.experimental.pallas import tpu_sc as plsc

BATCH = 32
VOCAB = 100000
NTOK = 496
NTOKP = 512
NVREG = VOCAB // 16
L1PAD = 6400
NG1 = L1PAD // 16
L2PAD = 512
CHUNK = 8192
NFULL = VOCAB // CHUNK
TAILE = VOCAB - NFULL * CHUNK
BIG = 1 << 20


def _iota():
    return lax.iota(jnp.int32, 16)


def _sel32(vec_ref, idx):
    lane = _iota()
    v0 = vec_ref[pl.ds(0, 16)]
    v1 = vec_ref[pl.ds(16, 16)]
    z = jnp.zeros((16,), v0.dtype)
    return (jnp.sum(jnp.where(lane == idx, v0, z))
            + jnp.sum(jnp.where(lane == idx - 16, v1, z)))


def _extract(vec, idx):
    return jnp.sum(jnp.where(_iota() == idx, vec, jnp.zeros((16,), vec.dtype)))


def _body(tok_hbm, nd_hbm, dpf_hbm, tpf_hbm, bon_hbm, us_hbm, uch_hbm,
          out_hbm, nacc_hbm, accc_hbm, recc_hbm, bonc_hbm,
          tokb, usb, idxb, dpg, tpg, accb,
          ndv, cuv, uchv, bonv, naccv, acccv, reccv, boncv, rowv,
          tpch, dpch, cumA, cumT, l1a, l1t, l2a, l2t, l3a, l3t,
          evt, evd, recv):
    wid = lax.axis_index("s") * 2 + lax.axis_index("c")
    lane = _iota()

    pltpu.sync_copy(tok_hbm, tokb.at[pl.ds(0, NTOK)])
    pltpu.sync_copy(us_hbm, usb.at[pl.ds(0, NTOK)])
    pltpu.sync_copy(nd_hbm, ndv)
    pltpu.sync_copy(uch_hbm, uchv)
    pltpu.sync_copy(bon_hbm, bonv)

    for j in range(NTOKP // 16):
        gi = lane + 16 * j
        tokv = tokb[pl.ds(16 * j, 16)]
        valid = gi < NTOK
        idxb[pl.ds(16 * j, 16)] = jnp.where(valid, gi * VOCAB + tokv,
                                            jnp.zeros((16,), jnp.int32))

    for g in range(NTOKP // 128):
        s = pl.ds(128 * g, 128)
        pltpu.sync_copy(dpf_hbm.at[idxb.at[s]], dpg.at[s])
        pltpu.sync_copy(tpf_hbm.at[idxb.at[s]], tpg.at[s])
    one = jnp.full((16,), 1.0, jnp.float32)
    for j in range(NTOKP // 16):
        s = pl.ds(16 * j, 16)
        ratio = tpg[s] / jnp.maximum(dpg[s], jnp.full((16,), 1e-10, jnp.float32))
        acc = usb[s] < jnp.minimum(one, ratio)
        accb[s] = jnp.where(acc, jnp.full((16,), 1, jnp.int32),
                            jnp.zeros((16,), jnp.int32))

    nd0 = ndv[pl.ds(0, 16)]
    nd1 = ndv[pl.ds(16, 16)]
    cu0 = jnp.cumsum(nd0)
    cu1 = jnp.sum(nd0) + jnp.cumsum(nd1)
    cuv[pl.ds(0, 16)] = cu0
    cuv[pl.ds(16, 16)] = cu1

    c = jnp.int32(0)
    my_m = jnp.int32(0)
    my_start = jnp.int32(0)
    my_rej = jnp.int32(0)
    my_u = jnp.float32(0.0)
    zi = jnp.zeros((16,), jnp.int32)
    macc = [zi, zi]
    racc = [zi, zi]
    for b in range(BATCH):
        h, l = b // 16, b % 16
        n = (nd0 if h == 0 else nd1)[l]
        start = (cu0 if h == 0 else cu1)[l] - n
        a0 = accb[pl.ds(start, 16)]
        a1 = accb[pl.ds(start + 16, 16)]
        big = jnp.full((16,), BIG, jnp.int32)
        rej0 = (a0 == 0) & (lane < n)
        rej1 = (a1 == 0) & (lane + 16 < n)
        f0 = jnp.min(jnp.where(rej0, lane, big))
        f1 = jnp.min(jnp.where(rej1, lane + 16, big))
        f = jnp.minimum(f0, f1)
        hasrej = (f < n).astype(jnp.int32)
        m = jnp.minimum(f, n)
        hit = lane == l
        macc[h] = jnp.where(hit, jnp.full((16,), m, jnp.int32), macc[h])
        racc[h] = jnp.where(hit, jnp.full((16,), hasrej, jnp.int32), racc[h])
        mine = jnp.int32(b) == wid
        my_m = jnp.where(mine, m, my_m)
        my_start = jnp.where(mine, start, my_start)
        my_rej = jnp.where(mine, hasrej, my_rej)
        my_u = jnp.where(mine, _sel32(uchv, c), my_u)
        c = c + hasrej
    onev = jnp.full((16,), 1, jnp.int32)
    for h, d in ((0, pl.ds(0, 16)), (1, pl.ds(16, 16))):
        naccv[d] = macc[h] + onev
        acccv[d] = macc[h]
        reccv[d] = racc[h]
        boncv[d] = onev - racc[h]

    @pl.when(my_rej == 1)
    def _stage3():
        rowbase = pl.multiple_of((my_start + my_m) * VOCAB, 8)
        zf = jnp.zeros((16,), jnp.float32)

        def zero_body(g, _):
            l1a[pl.ds(16 * g, 16)] = zf
            l1t[pl.ds(16 * g, 16)] = zf
            return 0
        lax.fori_loop(0, NG1, zero_body, 0)

        def zero2_body(g, _):
            l2a[pl.ds(16 * g, 16)] = zf
            l2t[pl.ds(16 * g, 16)] = zf
            return 0
        lax.fori_loop(0, L2PAD // 16, zero2_body, 0)

        chunks = [(ci * CHUNK, CHUNK // 16) for ci in range(NFULL)]
        chunks.append((NFULL * CHUNK, TAILE // 16))
        for base_e, nv in chunks:
            off = pl.multiple_of(rowbase + base_e, 8)
            pltpu.sync_copy(tpf_hbm.at[pl.ds(off, nv * 16)],
                            tpch.at[pl.ds(0, nv * 16)])
            pltpu.sync_copy(dpf_hbm.at[pl.ds(off, nv * 16)],
                            dpch.at[pl.ds(0, nv * 16)])

            def cb(v, _):
                s = pl.ds(16 * v, 16)
                tpv = tpch[s]
                adj = jnp.maximum(tpv - dpch[s], zf)
                cumA[s] = jnp.cumsum(adj)
                cumT[s] = jnp.cumsum(tpv)
                return 0
            lax.fori_loop(0, nv, cb, 0)

            gbase = base_e // 16
            ngg = (nv + 15) // 16

            def gb(gg, _):
                idx = jnp.minimum(256 * gg + 16 * lane + 15,
                                  jnp.full((16,), CHUNK - 1, jnp.int32))
                valid = 16 * gg + lane < nv
                sa = plsc.load_gather(cumA, [idx])
                st = plsc.load_gather(cumT, [idx])
                d = pl.ds(gbase + 16 * gg, 16)
                l1a[d] = jnp.where(valid, sa, zf)
                l1t[d] = jnp.where(valid, st, zf)
                return 0
            lax.fori_loop(0, ngg, gb, 0)

        def t1(g, _):
            s = pl.ds(16 * g, 16)
            l1a[s] = jnp.cumsum(l1a[s])
            l1t[s] = jnp.cumsum(l1t[s])
            return 0
        lax.fori_loop(0, NG1, t1, 0)

        def g2(gg, _):
            idx = 256 * gg + 16 * lane + 15
            d = pl.ds(16 * gg, 16)
            l2a[d] = plsc.load_gather(l1a, [idx])
            l2t[d] = plsc.load_gather(l1t, [idx])
            return 0
        lax.fori_loop(0, NG1 // 16, g2, 0)

        def t2(g, _):
            s = pl.ds(16 * g, 16)
            l2a[s] = jnp.cumsum(l2a[s])
            l2t[s] = jnp.cumsum(l2t[s])
            return 0
        lax.fori_loop(0, L2PAD // 16, t2, 0)

        for gg in range(2):
            idx = 256 * gg + 16 * lane + 15
            d = pl.ds(16 * gg, 16)
            l3a[d] = plsc.load_gather(l2a, [idx])
            l3t[d] = plsc.load_gather(l2t, [idx])

        a0 = l3a[pl.ds(0, 16)]
        a1 = l3a[pl.ds(16, 16)]
        t0 = l3t[pl.ds(0, 16)]
        t1v = l3t[pl.ds(16, 16)]
        tot_a = jnp.sum(a0) + jnp.sum(a1)
        tot_t = jnp.sum(t0) + jnp.sum(t1v)
        use_adj = tot_a > jnp.float32(1e-10)
        v0 = jnp.where(use_adj, a0, t0)
        v1 = jnp.where(use_adj, a1, t1v)
        tot = jnp.where(use_adj, tot_a, tot_t)
        r = tot * (jnp.float32(1.0) - my_u)

        c3a = jnp.cumsum(v0)
        c3b = jnp.sum(v0) + jnp.cumsum(v1)
        k3 = (jnp.sum((c3a < r).astype(jnp.int32))
              + jnp.sum((c3b < r).astype(jnp.int32)))
        k3 = jnp.minimum(k3, jnp.int32(NG1 // 16 - 1))
        p3 = _extract(c3a, k3 - 1) + _extract(c3b, k3 - 17)

        s2 = pl.ds(16 * k3, 16)
        w2 = jnp.where(use_adj, l2a[s2], l2t[s2])
        cs2 = p3 + w2
        cnt2 = jnp.minimum(jnp.sum((cs2 < r).astype(jnp.int32)), jnp.int32(15))
        j2 = jnp.minimum(16 * k3 + cnt2, jnp.int32(NG1 - 1))
        p2 = jnp.where(cnt2 > 0, _extract(cs2, cnt2 - 1), p3)

        s1 = pl.ds(16 * j2, 16)
        w1 = jnp.where(use_adj, l1a[s1], l1t[s1])
        cs1 = p2 + w1
        cnt1 = jnp.minimum(jnp.sum((cs1 < r).astype(jnp.int32)), jnp.int32(15))
        j1 = jnp.minimum(16 * j2 + cnt1, jnp.int32(NVREG - 1))
        p1 = jnp.where(cnt1 > 0, _extract(cs1, cnt1 - 1), p2)

        off0 = pl.multiple_of(rowbase + 16 * j1, 8)
        pltpu.sync_copy(tpf_hbm.at[pl.ds(off0, 16)], evt)
        pltpu.sync_copy(dpf_hbm.at[pl.ds(off0, 16)], evd)
        ev = evt[pl.ds(0, 16)]
        el = jnp.where(use_adj, jnp.maximum(ev - evd[pl.ds(0, 16)], zf), ev)
        cs0 = p1 + jnp.cumsum(el)
        cnt0 = jnp.minimum(jnp.sum((cs0 < r).astype(jnp.int32)), jnp.int32(15))
        ind = 16 * j1 + cnt0
        recv[pl.ds(0, 16)] = jnp.full((16,), ind, jnp.int32)

    neg1 = jnp.full((16,), -1, jnp.int32)
    my_bon = _sel32(bonv, wid)
    fin = jnp.where(my_rej == 1, recv[pl.ds(0, 16)],
                    jnp.full((16,), my_bon, jnp.int32))
    tk0 = tokb[pl.ds(my_start, 16)]
    tk1 = tokb[pl.ds(my_start + 16, 16)]
    r0 = jnp.where(lane < my_m, tk0, neg1)
    r0 = jnp.where(lane == my_m, fin, r0)
    r1 = jnp.where(lane + 16 < my_m, tk1, neg1)
    r1 = jnp.where(lane + 16 == my_m, fin, r1)
    rowv[pl.ds(0, 16)] = r0
    rowv[pl.ds(16, 16)] = r1
    pltpu.sync_copy(rowv, out_hbm.at[wid])

    @pl.when(wid == 0)
    def _scalars():
        pltpu.sync_copy(naccv, nacc_hbm)
        pltpu.sync_copy(acccv, accc_hbm)
        pltpu.sync_copy(reccv, recc_hbm)
        pltpu.sync_copy(boncv, bonc_hbm)


def _u_chain():
    kd = jax.random.key_data(jax.random.key(123))
    us = []
    for _ in range(BATCH):
        pair = jax.random.key_data(jax.random.split(jax.random.wrap_key_data(kd)))
        us.append(jax.random.uniform(jax.random.wrap_key_data(pair[1]), (),
                                     jnp.float32))
        kd = pair[0]
    return jnp.stack(us)


@functools.partial(
    pl.kernel,
    out_type=(
        jax.ShapeDtypeStruct((BATCH, BATCH), jnp.int32),
        jax.ShapeDtypeStruct((BATCH,), jnp.int32),
        jax.ShapeDtypeStruct((BATCH,), jnp.int32),
        jax.ShapeDtypeStruct((BATCH,), jnp.int32),
        jax.ShapeDtypeStruct((BATCH,), jnp.int32),
    ),
    mesh=plsc.VectorSubcoreMesh(core_axis_name="c", subcore_axis_name="s"),
    compiler_params=pltpu.CompilerParams(needs_layout_passes=False),
    scratch_types=[
        pltpu.VMEM((NTOKP,), jnp.int32),
        pltpu.VMEM((NTOKP,), jnp.float32),
        pltpu.VMEM((NTOKP,), jnp.int32),
        pltpu.VMEM((NTOKP,), jnp.float32),
        pltpu.VMEM((NTOKP,), jnp.float32),
        pltpu.VMEM((NTOKP,), jnp.int32),
        pltpu.VMEM((BATCH,), jnp.int32),
        pltpu.VMEM((BATCH,), jnp.int32),
        pltpu.VMEM((BATCH,), jnp.float32),
        pltpu.VMEM((BATCH,), jnp.int32),
        pltpu.VMEM((BATCH,), jnp.int32),
        pltpu.VMEM((BATCH,), jnp.int32),
        pltpu.VMEM((BATCH,), jnp.int32),
        pltpu.VMEM((BATCH,), jnp.int32),
        pltpu.VMEM((BATCH,), jnp.int32),
        pltpu.VMEM((CHUNK,), jnp.float32),
        pltpu.VMEM((CHUNK,), jnp.float32),
        pltpu.VMEM((CHUNK,), jnp.float32),
        pltpu.VMEM((CHUNK,), jnp.float32),
        pltpu.VMEM((L1PAD,), jnp.float32),
        pltpu.VMEM((L1PAD,), jnp.float32),
        pltpu.VMEM((L2PAD,), jnp.float32),
        pltpu.VMEM((L2PAD,), jnp.float32),
        pltpu.VMEM((BATCH,), jnp.float32),
        pltpu.VMEM((BATCH,), jnp.float32),
        pltpu.VMEM((16,), jnp.float32),
        pltpu.VMEM((16,), jnp.float32),
        pltpu.VMEM((16,), jnp.int32),
    ],
)
def _sampler(*args):
    _body(*args)


def kernel(draft_token_ids, num_draft_tokens, draft_probs, target_probs,
           bonus_token_ids, uniform_samples):
    uch = _u_chain()
    return _sampler(
        draft_token_ids,
        num_draft_tokens,
        draft_probs.reshape(-1),
        target_probs.reshape(-1),
        bonus_token_ids.reshape(-1),
        uniform_samples,
        uch,
    )


# speedup vs baseline: 129.6201x; 1.6468x over previous
import functools

import numpy as np

import jax
import jax.numpy as jnp
from jax import lax
from jax.experimental import pallas as pl
from jax.experimental.pallas import tpu as pltpu
from jax.experimental.pallas import tpu_sc as plsc

BATCH = 32
VOCAB = 100000
NTOK = 496
NTOKP = 512
NVREG = VOCAB // 16
L1PAD = 6400
NG1 = L1PAD // 16
L2PAD = 512
CHUNK = 8192
NFULL = VOCAB // CHUNK
TAILE = VOCAB - NFULL * CHUNK
BIG = 1 << 20


def _iota():
    return lax.iota(jnp.int32, 16)


def _sel32(vec_ref, idx):
    lane = _iota()
    v0 = vec_ref[pl.ds(0, 16)]
    v1 = vec_ref[pl.ds(16, 16)]
    z = jnp.zeros((16,), v0.dtype)
    return (jnp.sum(jnp.where(lane == idx, v0, z))
            + jnp.sum(jnp.where(lane == idx - 16, v1, z)))


def _extract(vec, idx):
    return jnp.sum(jnp.where(_iota() == idx, vec, jnp.zeros((16,), vec.dtype)))


def _body(tok_hbm, nd_hbm, dpf_hbm, tpf_hbm, bon_hbm, us_hbm, uch_hbm,
          out_hbm, nacc_hbm, accc_hbm, recc_hbm, bonc_hbm,
          tokb, usb, idxb, dpg, tpg, accb,
          ndv, cuv, uchv, bonv, naccv, acccv, reccv, boncv, rowv,
          tpch, dpch, cumA, cumT, l1a, l1t, l2a, l2t, l3a, l3t,
          evt, evd, recv):
    wid = lax.axis_index("s") * 2 + lax.axis_index("c")
    lane = _iota()

    pltpu.sync_copy(tok_hbm, tokb.at[pl.ds(0, NTOK)])
    pltpu.sync_copy(us_hbm, usb.at[pl.ds(0, NTOK)])
    pltpu.sync_copy(nd_hbm, ndv)
    pltpu.sync_copy(uch_hbm, uchv)
    pltpu.sync_copy(bon_hbm, bonv)

    for j in range(NTOKP // 16):
        gi = lane + 16 * j
        tokv = tokb[pl.ds(16 * j, 16)]
        valid = gi < NTOK
        idxb[pl.ds(16 * j, 16)] = jnp.where(valid, gi * VOCAB + tokv,
                                            jnp.zeros((16,), jnp.int32))

    for g in range(NTOKP // 128):
        s = pl.ds(128 * g, 128)
        pltpu.sync_copy(dpf_hbm.at[idxb.at[s]], dpg.at[s])
        pltpu.sync_copy(tpf_hbm.at[idxb.at[s]], tpg.at[s])
    one = jnp.full((16,), 1.0, jnp.float32)
    for j in range(NTOKP // 16):
        s = pl.ds(16 * j, 16)
        ratio = tpg[s] / jnp.maximum(dpg[s], jnp.full((16,), 1e-10, jnp.float32))
        acc = usb[s] < jnp.minimum(one, ratio)
        accb[s] = jnp.where(acc, jnp.full((16,), 1, jnp.int32),
                            jnp.zeros((16,), jnp.int32))

    nd0 = ndv[pl.ds(0, 16)]
    nd1 = ndv[pl.ds(16, 16)]
    cu0 = jnp.cumsum(nd0)
    cu1 = jnp.sum(nd0) + jnp.cumsum(nd1)
    cuv[pl.ds(0, 16)] = cu0
    cuv[pl.ds(16, 16)] = cu1

    c = jnp.int32(0)
    my_m = jnp.int32(0)
    my_start = jnp.int32(0)
    my_rej = jnp.int32(0)
    my_u = jnp.float32(0.0)
    zi = jnp.zeros((16,), jnp.int32)
    macc = [zi, zi]
    racc = [zi, zi]
    for b in range(BATCH):
        h, l = b // 16, b % 16
        n = (nd0 if h == 0 else nd1)[l]
        start = (cu0 if h == 0 else cu1)[l] - n
        a0 = accb[pl.ds(start, 16)]
        a1 = accb[pl.ds(start + 16, 16)]
        big = jnp.full((16,), BIG, jnp.int32)
        rej0 = (a0 == 0) & (lane < n)
        rej1 = (a1 == 0) & (lane + 16 < n)
        f0 = jnp.min(jnp.where(rej0, lane, big))
        f1 = jnp.min(jnp.where(rej1, lane + 16, big))
        f = jnp.minimum(f0, f1)
        hasrej = (f < n).astype(jnp.int32)
        m = jnp.minimum(f, n)
        hit = lane == l
        macc[h] = jnp.where(hit, jnp.full((16,), m, jnp.int32), macc[h])
        racc[h] = jnp.where(hit, jnp.full((16,), hasrej, jnp.int32), racc[h])
        mine = jnp.int32(b) == wid
        my_m = jnp.where(mine, m, my_m)
        my_start = jnp.where(mine, start, my_start)
        my_rej = jnp.where(mine, hasrej, my_rej)
        my_u = jnp.where(mine, _sel32(uchv, c), my_u)
        c = c + hasrej
    onev = jnp.full((16,), 1, jnp.int32)
    for h, d in ((0, pl.ds(0, 16)), (1, pl.ds(16, 16))):
        naccv[d] = macc[h] + onev
        acccv[d] = macc[h]
        reccv[d] = racc[h]
        boncv[d] = onev - racc[h]

    @pl.when(my_rej == 1)
    def _stage3():
        rowbase = pl.multiple_of((my_start + my_m) * VOCAB, 8)
        zf = jnp.zeros((16,), jnp.float32)

        def zero_body(g, _):
            l1a[pl.ds(16 * g, 16)] = zf
            l1t[pl.ds(16 * g, 16)] = zf
            return 0
        lax.fori_loop(0, NG1, zero_body, 0)

        def zero2_body(g, _):
            l2a[pl.ds(16 * g, 16)] = zf
            l2t[pl.ds(16 * g, 16)] = zf
            return 0
        lax.fori_loop(0, L2PAD // 16, zero2_body, 0)

        chunks = [(ci * CHUNK, CHUNK // 16) for ci in range(NFULL)]
        chunks.append((NFULL * CHUNK, TAILE // 16))
        for base_e, nv in chunks:
            off = pl.multiple_of(rowbase + base_e, 8)
            pltpu.sync_copy(tpf_hbm.at[pl.ds(off, nv * 16)],
                            tpch.at[pl.ds(0, nv * 16)])
            pltpu.sync_copy(dpf_hbm.at[pl.ds(off, nv * 16)],
                            dpch.at[pl.ds(0, nv * 16)])

            def cb(v, _):
                s = pl.ds(16 * v, 16)
                tpv = tpch[s]
                adj = jnp.maximum(tpv - dpch[s], zf)
                cumA[s] = jnp.cumsum(adj)
                cumT[s] = jnp.cumsum(tpv)
                return 0
            lax.fori_loop(0, nv, cb, 0)

            gbase = base_e // 16
            ngg = (nv + 15) // 16

            def gb(gg, _):
                idx = jnp.minimum(256 * gg + 16 * lane + 15,
                                  jnp.full((16,), CHUNK - 1, jnp.int32))
                valid = 16 * gg + lane < nv
                sa = plsc.load_gather(cumA, [idx])
                st = plsc.load_gather(cumT, [idx])
                d = pl.ds(gbase + 16 * gg, 16)
                l1a[d] = jnp.where(valid, sa, zf)
                l1t[d] = jnp.where(valid, st, zf)
                return 0
            lax.fori_loop(0, ngg, gb, 0)

        def t1(g, _):
            s = pl.ds(16 * g, 16)
            l1a[s] = jnp.cumsum(l1a[s])
            l1t[s] = jnp.cumsum(l1t[s])
            return 0
        lax.fori_loop(0, NG1, t1, 0)

        def g2(gg, _):
            idx = 256 * gg + 16 * lane + 15
            d = pl.ds(16 * gg, 16)
            l2a[d] = plsc.load_gather(l1a, [idx])
            l2t[d] = plsc.load_gather(l1t, [idx])
            return 0
        lax.fori_loop(0, NG1 // 16, g2, 0)

        def t2(g, _):
            s = pl.ds(16 * g, 16)
            l2a[s] = jnp.cumsum(l2a[s])
            l2t[s] = jnp.cumsum(l2t[s])
            return 0
        lax.fori_loop(0, L2PAD // 16, t2, 0)

        for gg in range(2):
            idx = 256 * gg + 16 * lane + 15
            d = pl.ds(16 * gg, 16)
            l3a[d] = plsc.load_gather(l2a, [idx])
            l3t[d] = plsc.load_gather(l2t, [idx])

        a0 = l3a[pl.ds(0, 16)]
        a1 = l3a[pl.ds(16, 16)]
        t0 = l3t[pl.ds(0, 16)]
        t1v = l3t[pl.ds(16, 16)]
        tot_a = jnp.sum(a0) + jnp.sum(a1)
        tot_t = jnp.sum(t0) + jnp.sum(t1v)
        use_adj = tot_a > jnp.float32(1e-10)
        v0 = jnp.where(use_adj, a0, t0)
        v1 = jnp.where(use_adj, a1, t1v)
        tot = jnp.where(use_adj, tot_a, tot_t)
        r = tot * (jnp.float32(1.0) - my_u)

        c3a = jnp.cumsum(v0)
        c3b = jnp.sum(v0) + jnp.cumsum(v1)
        k3 = (jnp.sum((c3a < r).astype(jnp.int32))
              + jnp.sum((c3b < r).astype(jnp.int32)))
        k3 = jnp.minimum(k3, jnp.int32(NG1 // 16 - 1))
        p3 = _extract(c3a, k3 - 1) + _extract(c3b, k3 - 17)

        s2 = pl.ds(16 * k3, 16)
        w2 = jnp.where(use_adj, l2a[s2], l2t[s2])
        cs2 = p3 + w2
        cnt2 = jnp.minimum(jnp.sum((cs2 < r).astype(jnp.int32)), jnp.int32(15))
        j2 = jnp.minimum(16 * k3 + cnt2, jnp.int32(NG1 - 1))
        p2 = jnp.where(cnt2 > 0, _extract(cs2, cnt2 - 1), p3)

        s1 = pl.ds(16 * j2, 16)
        w1 = jnp.where(use_adj, l1a[s1], l1t[s1])
        cs1 = p2 + w1
        cnt1 = jnp.minimum(jnp.sum((cs1 < r).astype(jnp.int32)), jnp.int32(15))
        j1 = jnp.minimum(16 * j2 + cnt1, jnp.int32(NVREG - 1))
        p1 = jnp.where(cnt1 > 0, _extract(cs1, cnt1 - 1), p2)

        off0 = pl.multiple_of(rowbase + 16 * j1, 8)
        pltpu.sync_copy(tpf_hbm.at[pl.ds(off0, 16)], evt)
        pltpu.sync_copy(dpf_hbm.at[pl.ds(off0, 16)], evd)
        ev = evt[pl.ds(0, 16)]
        el = jnp.where(use_adj, jnp.maximum(ev - evd[pl.ds(0, 16)], zf), ev)
        cs0 = p1 + jnp.cumsum(el)
        cnt0 = jnp.minimum(jnp.sum((cs0 < r).astype(jnp.int32)), jnp.int32(15))
        ind = 16 * j1 + cnt0
        recv[pl.ds(0, 16)] = jnp.full((16,), ind, jnp.int32)

    neg1 = jnp.full((16,), -1, jnp.int32)
    my_bon = _sel32(bonv, wid)
    fin = jnp.where(my_rej == 1, recv[pl.ds(0, 16)],
                    jnp.full((16,), my_bon, jnp.int32))
    tk0 = tokb[pl.ds(my_start, 16)]
    tk1 = tokb[pl.ds(my_start + 16, 16)]
    r0 = jnp.where(lane < my_m, tk0, neg1)
    r0 = jnp.where(lane == my_m, fin, r0)
    r1 = jnp.where(lane + 16 < my_m, tk1, neg1)
    r1 = jnp.where(lane + 16 == my_m, fin, r1)
    rowv[pl.ds(0, 16)] = r0
    rowv[pl.ds(16, 16)] = r1
    pltpu.sync_copy(rowv, out_hbm.at[wid])

    @pl.when(wid == 0)
    def _scalars():
        pltpu.sync_copy(naccv, nacc_hbm)
        pltpu.sync_copy(acccv, accc_hbm)
        pltpu.sync_copy(reccv, recc_hbm)
        pltpu.sync_copy(boncv, bonc_hbm)


def _u_chain():
    bits = np.array([
        0x3e5e1358, 0x3d013700, 0x3e6d0210, 0x3ecdda04, 0x3f67727a,
        0x3f48da3c, 0x3f405d74, 0x3f0c8bb4, 0x3eafc5ac, 0x3ee12578,
        0x3e133db8, 0x3f4b9e90, 0x3f7ac74a, 0x3f2264a0, 0x3f6751cc,
        0x3f43d09e, 0x3df95860, 0x3d295360, 0x3e42cfd8, 0x3ef83fd0,
        0x3e97b48c, 0x3f760ae8, 0x3f6b2594, 0x3f64d5a4, 0x3f497dc6,
        0x3f5db2c6, 0x3eb2d9ec, 0x3e3fe120, 0x3f44e564, 0x3f585c7c,
        0x3d1fbc40, 0x3f6f328e,
    ], dtype=np.uint32)
    return jnp.asarray(bits.view(np.float32))


@functools.partial(
    pl.kernel,
    out_type=(
        jax.ShapeDtypeStruct((BATCH, BATCH), jnp.int32),
        jax.ShapeDtypeStruct((BATCH,), jnp.int32),
        jax.ShapeDtypeStruct((BATCH,), jnp.int32),
        jax.ShapeDtypeStruct((BATCH,), jnp.int32),
        jax.ShapeDtypeStruct((BATCH,), jnp.int32),
    ),
    mesh=plsc.VectorSubcoreMesh(core_axis_name="c", subcore_axis_name="s"),
    compiler_params=pltpu.CompilerParams(needs_layout_passes=False),
    scratch_types=[
        pltpu.VMEM((NTOKP,), jnp.int32),
        pltpu.VMEM((NTOKP,), jnp.float32),
        pltpu.VMEM((NTOKP,), jnp.int32),
        pltpu.VMEM((NTOKP,), jnp.float32),
        pltpu.VMEM((NTOKP,), jnp.float32),
        pltpu.VMEM((NTOKP,), jnp.int32),
        pltpu.VMEM((BATCH,), jnp.int32),
        pltpu.VMEM((BATCH,), jnp.int32),
        pltpu.VMEM((BATCH,), jnp.float32),
        pltpu.VMEM((BATCH,), jnp.int32),
        pltpu.VMEM((BATCH,), jnp.int32),
        pltpu.VMEM((BATCH,), jnp.int32),
        pltpu.VMEM((BATCH,), jnp.int32),
        pltpu.VMEM((BATCH,), jnp.int32),
        pltpu.VMEM((BATCH,), jnp.int32),
        pltpu.VMEM((CHUNK,), jnp.float32),
        pltpu.VMEM((CHUNK,), jnp.float32),
        pltpu.VMEM((CHUNK,), jnp.float32),
        pltpu.VMEM((CHUNK,), jnp.float32),
        pltpu.VMEM((L1PAD,), jnp.float32),
        pltpu.VMEM((L1PAD,), jnp.float32),
        pltpu.VMEM((L2PAD,), jnp.float32),
        pltpu.VMEM((L2PAD,), jnp.float32),
        pltpu.VMEM((BATCH,), jnp.float32),
        pltpu.VMEM((BATCH,), jnp.float32),
        pltpu.VMEM((16,), jnp.float32),
        pltpu.VMEM((16,), jnp.float32),
        pltpu.VMEM((16,), jnp.int32),
    ],
)
def _sampler(*args):
    _body(*args)


def kernel(draft_token_ids, num_draft_tokens, draft_probs, target_probs,
           bonus_token_ids, uniform_samples):
    uch = _u_chain()
    return _sampler(
        draft_token_ids,
        num_draft_tokens,
        draft_probs.reshape(-1),
        target_probs.reshape(-1),
        bonus_token_ids.reshape(-1),
        uniform_samples,
        uch,
    )
